# Initial kernel scaffold; baseline (speedup 1.0000x reference)
import jax
import jax.numpy as jnp
from jax.experimental import pallas as pl


def kernel(x, W1, b1, W2, b2, W3, b3, W4, b4, cb1, cb2):
    raise NotImplementedError("write your pallas kernel here")


# speedup vs baseline: 8.3785x; 8.3785x over previous
import jax
import jax.numpy as jnp
from jax.experimental import pallas as pl

EMB_DIM = 64
NUM_EMB = 8192
CC = 0.25

_ROWS = 1024
_WIN = 2048


def _conv1d(x, w, b, stride, pad):
    y = jax.lax.conv_general_dilated(
        x, w, window_strides=(stride,), padding=[(pad, pad)],
        dimension_numbers=('NCH', 'OIH', 'NCH'))
    return y + b[None, :, None]


def _argmin_body(flat_ref, fsq_ref, cb_ref, idx_ref):
    flat = flat_ref[...]
    flat_sq = fsq_ref[...]
    fb = flat.astype(jnp.bfloat16)
    v = jnp.full((_ROWS, 1), jnp.inf, jnp.float32)
    a = jnp.zeros((_ROWS, 1), jnp.int32)
    for w in range(NUM_EMB // _WIN):
        cb_w = cb_ref[pl.ds(w * _WIN, _WIN), :]
        cb_sq = jnp.sum(cb_w * cb_w, axis=1)
        mm = jax.lax.dot_general(
            fb, cb_w.astype(jnp.bfloat16), (((1,), (1,)), ((), ())),
            preferred_element_type=jnp.float32)
        d = (flat_sq + cb_sq[None, :]) - 2.0 * mm
        m_w = jnp.min(d, axis=1, keepdims=True)
        lane = jax.lax.broadcasted_iota(jnp.int32, d.shape, 1)
        a_w = jnp.min(jnp.where(d == m_w, lane, NUM_EMB),
                      axis=1, keepdims=True) + w * _WIN
        upd = m_w < v
        v = jnp.where(upd, m_w, v)
        a = jnp.where(upd, a_w, a)
        v = v.astype(jnp.bfloat16).astype(jnp.float32)
    idx_ref[...] = a


def _vq_argmin(flat, fsq, cb):
    n = flat.shape[0]
    return pl.pallas_call(
        _argmin_body,
        grid=(n // _ROWS,),
        in_specs=[
            pl.BlockSpec((_ROWS, EMB_DIM), lambda i: (i, 0)),
            pl.BlockSpec((_ROWS, 1), lambda i: (i, 0)),
            pl.BlockSpec((NUM_EMB, EMB_DIM), lambda i: (0, 0)),
        ],
        out_specs=pl.BlockSpec((_ROWS, 1), lambda i: (i, 0)),
        out_shape=jax.ShapeDtypeStruct((n, 1), jnp.int32),
    )(flat, fsq, cb)


def _quantize(inputs, cb):
    flat = inputs.reshape(-1, EMB_DIM)
    n = flat.shape[0]
    fsq = jnp.sum(flat ** 2, axis=1, keepdims=True)
    idx = _vq_argmin(flat, fsq, cb)[:, 0]
    q = cb[idx].astype(jnp.bfloat16).astype(jnp.float32)
    diff = q - flat
    mean_sq = jnp.sum(diff * diff) / (n * EMB_DIM)
    loss = mean_sq + CC * mean_sq
    z_q = inputs + (q.reshape(inputs.shape) - inputs)
    return z_q, loss


def kernel(x, W1, b1, W2, b2, W3, b3, W4, b4, cb1, cb2):
    h = jax.nn.relu(_conv1d(x, W1, b1, 2, 1))
    h = jax.nn.relu(_conv1d(h, W2, b2, 2, 1))
    z_e1 = _conv1d(h, W3, b3, 2, 1)
    z_q1, vq_loss1 = _quantize(z_e1, cb1)
    z_e2 = jax.nn.relu(_conv1d(z_q1, W4, b4, 1, 1))
    z_q2, vq_loss2 = _quantize(z_e2, cb2)
    return (z_q2, vq_loss1, vq_loss2)
